# baseline (speedup 1.0000x reference)
import functools

import jax
import jax.numpy as jnp
from jax import lax
from jax.experimental import pallas as pl
from jax.experimental.pallas import tpu as pltpu
from jax.experimental.pallas import tpu_sc as plsc

VOCAB = 49408
NUM_POS = 77
EMBED = 768
BATCH = 1024

L = 16
NC = 2
NS = 16
NW = NC * NS
BATCH_PER_W = BATCH // NW
ROWS_PER_W = BATCH_PER_W * NUM_POS
CHUNK = 11
NCHUNK = ROWS_PER_W // CHUNK
NBUF = 7
DEPTH = NBUF - 2


def _emb_body(idx_hbm, tok_hbm, pos_hbm, out_hbm,
              idx_v, pos_v,
              b0, b1, b2, b3, b4, b5, b6,
              g0, g1, g2, g3, g4, g5, g6,
              s0, s1, s2, s3, s4, s5, s6):
    bufs = (b0, b1, b2, b3, b4, b5, b6)
    gsem = (g0, g1, g2, g3, g4, g5, g6)
    ssem = (s0, s1, s2, s3, s4, s5, s6)

    wid = lax.axis_index("s") * NC + lax.axis_index("c")
    row0 = wid * ROWS_PER_W

    pltpu.sync_copy(idx_hbm.at[wid], idx_v)
    pltpu.sync_copy(pos_hbm, pos_v)

    def gather(c, b):
        return pltpu.make_async_copy(tok_hbm.at[idx_v.at[c]], bufs[b], gsem[b])

    def scatter(c, b):
        return pltpu.make_async_copy(
            bufs[b], out_hbm.at[pl.ds(row0 + c * CHUNK, CHUNK)], ssem[b])

    for b in range(DEPTH):
        gather(b, b).start()

    def outer(t, carry):
        for j in range(NBUF):
            c = t * NBUF + j
            q = c + DEPTH
            bq = (j + DEPTH) % NBUF

            @pl.when(q < NCHUNK)
            def _():
                @pl.when(q >= NBUF)
                def _():
                    scatter(q - NBUF, bq).wait()
                gather(q, bq).start()

            gather(c, j).wait()

            def row_add(r, carry2):
                pr = j * CHUNK + r
                for k in range(EMBED // L):
                    sl = pl.ds(k * L, L)
                    plsc.addupdate(bufs[j].at[r, sl], pos_v[pr, sl])
                return carry2

            lax.fori_loop(0, CHUNK, row_add, 0, unroll=False)

            scatter(c, j).start()
        return carry

    lax.fori_loop(0, NCHUNK // NBUF, outer, 0, unroll=False)

    for j in range(NBUF):
        scatter(NCHUNK - NBUF + j, j).wait()


@jax.jit
def _emb_call(idx3, token_table, position_table):
    info = plsc.get_sparse_core_info()
    assert info.num_cores == NC and info.num_subcores == NS

    mesh = plsc.VectorSubcoreMesh(core_axis_name="c", subcore_axis_name="s")
    run = functools.partial(
        pl.kernel,
        mesh=mesh,
        compiler_params=pltpu.CompilerParams(use_tc_tiling_on_sc=False),
        out_type=jax.ShapeDtypeStruct((BATCH * NUM_POS, EMBED), jnp.float32),
        scratch_types=(
            [pltpu.VMEM((NCHUNK, CHUNK), jnp.int32),
             pltpu.VMEM((NUM_POS, EMBED), jnp.float32)]
            + [pltpu.VMEM((CHUNK, EMBED), jnp.float32)] * NBUF
            + [pltpu.SemaphoreType.DMA] * (2 * NBUF)
        ),
    )(_emb_body)
    return run(idx3, token_table, position_table)


def kernel(input_tokens, token_table, position_table):
    idx3 = input_tokens.astype(jnp.int32).reshape(NW, NCHUNK, CHUNK)
    out = _emb_call(idx3, token_table.astype(jnp.float32),
                    position_table.astype(jnp.float32))
    return out.reshape(BATCH, NUM_POS, EMBED)
